# Initial kernel scaffold; baseline (speedup 1.0000x reference)
import jax
import jax.numpy as jnp
from jax.experimental import pallas as pl


def kernel(question1, answer1, user1, score1, question_first_layer, question_edge_first_layer, question_edge_score_list_first_layer, question_second_layer, question_edge_second_layer, question_edge_score_list_second_layer, user_first_layer, user_edge_first_layer, user_edge_score_list_first_layer, user_second_layer, user_edge_second_layer, user_edge_score_list_second_layer, word_embed, user_embed, Wih, Whh, b_lstm, We, Ws, Wd, be, Wn_q, Wea_q, Wsa_q, b_agg_q, Wn_u, Wea_u, Wsa_u, b_agg_u, wq, wa, wu, bq, ba, bu, wfinal, bfinal):
    raise NotImplementedError("write your pallas kernel here")


# speedup vs baseline: 2.2327x; 2.2327x over previous
import functools

import jax
import jax.numpy as jnp
from jax import lax
from jax.experimental import pallas as pl
from jax.experimental.pallas import tpu as pltpu
from jax.experimental.pallas import tpu_sc as plsc

D = 128
H = 128
LSEQ = 24
B = 32
N1 = 12
N2 = 12

_NC, _NS = 2, 16
_NW = _NC * _NS
_CH = 128

_M_TOT = B * (2 + 3 * N1 + 3 * N1 * N2)
_MB = 1024
_M_PAD = 15360
_NTOK = _M_PAD * LSEQ
_NW_CHUNKS = _NTOK // (_NW * _CH)
_NUSR = B * (1 + N1 + N1 * N2)
_NUSR_PAD = _NW * 2 * _CH
_NU_CHUNKS = 2


def _sc_gather(word_ids, user_ids, word_tab, user_tab):
    mesh = plsc.VectorSubcoreMesh(core_axis_name="c", subcore_axis_name="s")

    @functools.partial(
        pl.kernel,
        out_type=(
            jax.ShapeDtypeStruct((_NTOK, D), jnp.float32),
            jax.ShapeDtypeStruct((_NUSR_PAD, D), jnp.float32),
        ),
        mesh=mesh,
        scratch_types=[
            pltpu.VMEM((_NW_CHUNKS * _CH,), jnp.int32),
            pltpu.VMEM((_NU_CHUNKS * _CH,), jnp.int32),
            pltpu.VMEM((_CH, D), jnp.float32),
            pltpu.SemaphoreType.DMA,
        ],
    )
    def k(wids, uids, wtab, utab, wout, uout, widx_v, uidx_v, rows_v, sem):
        w = lax.axis_index("s") * _NC + lax.axis_index("c")
        wbase = w * _NW_CHUNKS * _CH
        ubase = w * _NU_CHUNKS * _CH
        pltpu.sync_copy(wids.at[pl.ds(wbase, _NW_CHUNKS * _CH)], widx_v)
        pltpu.sync_copy(uids.at[pl.ds(ubase, _NU_CHUNKS * _CH)], uidx_v)

        def wbody(j, carry):
            pltpu.async_copy(
                wtab.at[widx_v.at[pl.ds(j * _CH, _CH)]], rows_v, sem).wait()
            pltpu.sync_copy(rows_v, wout.at[pl.ds(wbase + j * _CH, _CH)])
            return carry

        lax.fori_loop(0, _NW_CHUNKS, wbody, 0)

        def ubody(j, carry):
            pltpu.async_copy(
                utab.at[uidx_v.at[pl.ds(j * _CH, _CH)]], rows_v, sem).wait()
            pltpu.sync_copy(rows_v, uout.at[pl.ds(ubase + j * _CH, _CH)])
            return carry

        lax.fori_loop(0, _NU_CHUNKS, ubody, 0)

    return k(word_ids, user_ids, word_tab, user_tab)


def _lstm_maxpool_all(x, w_cat, b):
    m = x.shape[0]
    grid = m // _MB

    def body(x_ref, w_ref, b_ref, out_ref):
        wc = w_ref[...]
        bb = b_ref[...]
        h = jnp.zeros((_MB, H), jnp.float32)
        c = jnp.zeros((_MB, H), jnp.float32)
        hm = jnp.full((_MB, H), -jnp.inf, jnp.float32)
        for t in range(LSEQ):
            xt = x_ref[:, t, :]
            xh = jnp.concatenate([xt, h], axis=1)
            g = jnp.dot(xh, wc, preferred_element_type=jnp.float32) + bb
            i = jax.nn.sigmoid(g[:, :H])
            f = jax.nn.sigmoid(g[:, H:2 * H])
            gg = jnp.tanh(g[:, 2 * H:3 * H])
            o = jax.nn.sigmoid(g[:, 3 * H:])
            c = f * c + i * gg
            h = o * jnp.tanh(c)
            hm = jnp.maximum(hm, h)
        out_ref[...] = hm

    return pl.pallas_call(
        body,
        grid=(grid,),
        in_specs=[
            pl.BlockSpec((_MB, LSEQ, D), lambda i: (i, 0, 0)),
            pl.BlockSpec((D + H, 4 * H), lambda i: (0, 0)),
            pl.BlockSpec((1, 4 * H), lambda i: (0, 0)),
        ],
        out_specs=pl.BlockSpec((_MB, H), lambda i: (i, 0)),
        out_shape=jax.ShapeDtypeStruct((m, H), jnp.float32),
    )(x, w_cat, b)


def _second_layer(qsl3, qesl3, usl3, uesl3, qfl, ufl,
                  We, Ws, Wd, be, Wn_q, Wea_q, Wsa_q, b_agg_q,
                  Wn_u, Wea_u, Wsa_u, b_agg_u):
    r = qfl.shape[0]
    scale = 1.0 / float(H) ** 0.5

    def body(qsl_r, qesl_r, usl_r, uesl_r, qfl_r, ufl_r,
             we_r, ws_r, wd_r, be_r, wnq_r, weaq_r, wsaq_r, bq_r,
             wnu_r, weau_r, wsau_r, bu_r,
             qout_r, uout_r, msgq_s, msgu_s):
        dot = lambda a, m: jnp.dot(a, m, preferred_element_type=jnp.float32)
        qfl2 = qfl_r[...]
        ufl2 = ufl_r[...]
        we = we_r[...]
        ws = ws_r[...]
        wd = wd_r[...]
        be2 = be_r[...]
        ufl_wd = dot(ufl2, wd) + be2
        qfl_ws = dot(qfl2, ws) + be2
        wed = we + wd
        sq_list = []
        su_list = []
        for j in range(N2):
            qe = jnp.tanh(dot(qesl_r[:, j, :], we) + dot(qsl_r[:, j, :], ws)
                          + ufl_wd)
            ue = jnp.tanh(dot(uesl_r[:, j, :], wed) + qfl_ws)
            mq = jnp.tanh(dot(usl_r[:, j, :], wnq_r[...]) + dot(qe, weaq_r[...]))
            mu = jnp.tanh(dot(qsl_r[:, j, :], wnu_r[...]) + dot(ue, weau_r[...]))
            msgq_s[:, j, :] = mq
            msgu_s[:, j, :] = mu
            sq_list.append(jnp.sum(qfl2 * mq, axis=1, keepdims=True))
            su_list.append(jnp.sum(ufl2 * mu, axis=1, keepdims=True))

        def softmax(s):
            mx = jnp.max(s, axis=1, keepdims=True)
            e = jnp.exp(s - mx)
            return e / jnp.sum(e, axis=1, keepdims=True)

        aq = softmax(jnp.concatenate(sq_list, axis=1) * scale)
        au = softmax(jnp.concatenate(su_list, axis=1) * scale)
        aggq = jnp.zeros((r, H), jnp.float32)
        aggu = jnp.zeros((r, H), jnp.float32)
        for j in range(N2):
            aggq = aggq + aq[:, j:j + 1] * msgq_s[:, j, :]
            aggu = aggu + au[:, j:j + 1] * msgu_s[:, j, :]
        qout_r[...] = jnp.tanh(dot(qfl2, wsaq_r[...]) + aggq + bq_r[...])
        uout_r[...] = jnp.tanh(dot(ufl2, wsau_r[...]) + aggu + bu_r[...])

    return pl.pallas_call(
        body,
        out_shape=(jax.ShapeDtypeStruct((r, H), jnp.float32),
                   jax.ShapeDtypeStruct((r, H), jnp.float32)),
        scratch_shapes=[pltpu.VMEM((r, N2, H), jnp.float32),
                        pltpu.VMEM((r, N2, H), jnp.float32)],
    )(qsl3, qesl3, usl3, uesl3, qfl, ufl,
      We, Ws, Wd, be, Wn_q, Wea_q, Wsa_q, b_agg_q,
      Wn_u, Wea_u, Wsa_u, b_agg_u)


def _first_layer(qefl3, uefl3, ufl3, qfl3, q1, a1, u1,
                 We, Ws, Wd, be, Wn_q, Wea_q, Wsa_q, b_agg_q,
                 Wn_u, Wea_u, Wsa_u, b_agg_u,
                 wq, wa, wu, bq, ba, bu, wfinal, bfinal):
    scale = 1.0 / float(H) ** 0.5

    def body(qefl_r, uefl_r, ufl_r, qfl_r, q1_r, a1_r, u1_r,
             we_r, ws_r, wd_r, be_r, wnq_r, weaq_r, wsaq_r, bgq_r,
             wnu_r, weau_r, wsau_r, bgu_r,
             wq_r, wa_r, wu_r, bq_r, ba_r, bu_r, wf_r, bf_r,
             res_r, pred_r, msgq_s, msgu_s):
        dot = lambda a, m: jnp.dot(a, m, preferred_element_type=jnp.float32)
        q1v = q1_r[...]
        a1v = a1_r[...]
        u1v = u1_r[...]
        we = we_r[...]
        ws = ws_r[...]
        wd = wd_r[...]
        be2 = be_r[...]
        q1_ws = dot(q1v, ws) + be2
        u1_wd = dot(u1v, wd) + be2
        sq_list = []
        su_list = []
        for j in range(N1):
            uflj = ufl_r[:, j, :]
            qflj = qfl_r[:, j, :]
            qe = jnp.tanh(dot(qefl_r[:, j, :], we) + q1_ws + dot(uflj, wd))
            ue = jnp.tanh(dot(uefl_r[:, j, :], we) + dot(qflj, ws) + u1_wd)
            mq = jnp.tanh(dot(uflj, wnq_r[...]) + dot(qe, weaq_r[...]))
            mu = jnp.tanh(dot(qflj, wnu_r[...]) + dot(ue, weau_r[...]))
            msgq_s[:, j, :] = mq
            msgu_s[:, j, :] = mu
            sq_list.append(jnp.sum(q1v * mq, axis=1, keepdims=True))
            su_list.append(jnp.sum(u1v * mu, axis=1, keepdims=True))

        def softmax(s):
            mx = jnp.max(s, axis=1, keepdims=True)
            e = jnp.exp(s - mx)
            return e / jnp.sum(e, axis=1, keepdims=True)

        aq = softmax(jnp.concatenate(sq_list, axis=1) * scale)
        au = softmax(jnp.concatenate(su_list, axis=1) * scale)
        aggq = jnp.zeros((B, H), jnp.float32)
        aggu = jnp.zeros((B, H), jnp.float32)
        for j in range(N1):
            aggq = aggq + aq[:, j:j + 1] * msgq_s[:, j, :]
            aggu = aggu + au[:, j:j + 1] * msgu_s[:, j, :]
        q1n = jnp.tanh(dot(q1v, wsaq_r[...]) + aggq + bgq_r[...])
        u1n = jnp.tanh(dot(u1v, wsau_r[...]) + aggu + bgu_r[...])
        a1n = jnp.tanh(dot(a1v, we) + dot(q1n, ws) + dot(u1n, wd) + be2)
        t = jnp.tanh(dot(q1n, wq_r[...]) + bq_r[...]
                     + dot(a1n, wa_r[...]) + ba_r[...]
                     + dot(u1n, wu_r[...]) + bu_r[...])
        z = dot(t, wf_r[...]) + bf_r[...]
        mx = jnp.max(z, axis=1, keepdims=True)
        lse = mx + jnp.log(jnp.sum(jnp.exp(z - mx), axis=1, keepdims=True))
        res_r[...] = z - lse
        pred_r[...] = (z[:, 1:2] > z[:, 0:1]).astype(jnp.int32)

    return pl.pallas_call(
        body,
        out_shape=(jax.ShapeDtypeStruct((B, 2), jnp.float32),
                   jax.ShapeDtypeStruct((B, 1), jnp.int32)),
        scratch_shapes=[pltpu.VMEM((B, N1, H), jnp.float32),
                        pltpu.VMEM((B, N1, H), jnp.float32)],
    )(qefl3, uefl3, ufl3, qfl3, q1, a1, u1,
      We, Ws, Wd, be, Wn_q, Wea_q, Wsa_q, b_agg_q,
      Wn_u, Wea_u, Wsa_u, b_agg_u,
      wq, wa, wu, bq, ba, bu, wfinal, bfinal)


def kernel(question1, answer1, user1, score1, question_first_layer,
           question_edge_first_layer, question_edge_score_list_first_layer,
           question_second_layer, question_edge_second_layer,
           question_edge_score_list_second_layer, user_first_layer,
           user_edge_first_layer, user_edge_score_list_first_layer,
           user_second_layer, user_edge_second_layer,
           user_edge_score_list_second_layer, word_embed, user_embed,
           Wih, Whh, b_lstm, We, Ws, Wd, be, Wn_q, Wea_q, Wsa_q, b_agg_q,
           Wn_u, Wea_u, Wsa_u, b_agg_u, wq, wa, wu, bq, ba, bu,
           wfinal, bfinal):
    i32 = jnp.int32
    wids = jnp.concatenate([
        question1.reshape(-1, LSEQ), answer1.reshape(-1, LSEQ),
        user_edge_first_layer.reshape(-1, LSEQ),
        user_edge_second_layer.reshape(-1, LSEQ),
        question_first_layer.reshape(-1, LSEQ),
        question_edge_first_layer.reshape(-1, LSEQ),
        question_second_layer.reshape(-1, LSEQ),
        question_edge_second_layer.reshape(-1, LSEQ),
    ], axis=0).reshape(-1).astype(i32)
    wids = jnp.concatenate(
        [wids, jnp.zeros((_NTOK - _M_TOT * LSEQ,), i32)])
    uids = jnp.concatenate([
        user1.reshape(-1), user_first_layer.reshape(-1),
        user_second_layer.reshape(-1)]).astype(i32)
    uids = jnp.concatenate([uids, jnp.zeros((_NUSR_PAD - _NUSR,), i32)])

    xw, xu = _sc_gather(wids, uids, word_embed, user_embed)

    w_cat = jnp.concatenate([Wih, Whh], axis=0)
    hs = _lstm_maxpool_all(xw.reshape(_M_PAD, LSEQ, D), w_cat,
                           b_lstm.reshape(1, 4 * H))

    o = 0
    q1 = hs[o:o + B]; o += B
    a1 = hs[o:o + B]; o += B
    uefl = hs[o:o + B * N1]; o += B * N1
    uesl = hs[o:o + B * N1 * N2]; o += B * N1 * N2
    qfl = hs[o:o + B * N1]; o += B * N1
    qefl = hs[o:o + B * N1]; o += B * N1
    qsl = hs[o:o + B * N1 * N2]; o += B * N1 * N2
    qesl = hs[o:o + B * N1 * N2]; o += B * N1 * N2

    u1 = xu[:B]
    ufl = xu[B:B + B * N1]
    usl = xu[B + B * N1:_NUSR]

    r3 = lambda a: a.reshape(B * N1, N2, H)
    be2 = be.reshape(1, H)
    qfl_new, ufl_new = _second_layer(
        r3(qsl), r3(qesl), r3(usl), r3(uesl), qfl, ufl,
        We, Ws, Wd, be2, Wn_q, Wea_q, Wsa_q, b_agg_q.reshape(1, H),
        Wn_u, Wea_u, Wsa_u, b_agg_u.reshape(1, H))

    f3 = lambda a: a.reshape(B, N1, H)
    result, predict = _first_layer(
        f3(qefl), f3(uefl), f3(ufl_new), f3(qfl_new), q1, a1, u1,
        We, Ws, Wd, be2, Wn_q, Wea_q, Wsa_q, b_agg_q.reshape(1, H),
        Wn_u, Wea_u, Wsa_u, b_agg_u.reshape(1, H),
        wq, wa, wu, bq.reshape(1, H), ba.reshape(1, H), bu.reshape(1, H),
        wfinal, bfinal.reshape(1, 2))

    return (result, predict.reshape(B))


# speedup vs baseline: 2.3494x; 1.0523x over previous
import functools

import jax
import jax.numpy as jnp
from jax import lax
from jax.experimental import pallas as pl
from jax.experimental.pallas import tpu as pltpu
from jax.experimental.pallas import tpu_sc as plsc

D = 128
H = 128
LSEQ = 24
B = 32
N1 = 12
N2 = 12

_NC, _NS = 2, 16
_NW = _NC * _NS
_CH = 128

_M_TOT = B * (2 + 3 * N1 + 3 * N1 * N2)
_MB = 1024
_M_PAD = 15360
_NTOK = _M_PAD * LSEQ
_NW_CHUNKS = _NTOK // (_NW * _CH)
_NUSR = B * (1 + N1 + N1 * N2)
_NUSR_PAD = _NW * 2 * _CH
_NU_CHUNKS = 2


_NBUF = 6
_NGRP = _NW_CHUNKS // _NBUF


def _sc_gather(word_ids, user_ids, word_tab, user_tab):
    mesh = plsc.VectorSubcoreMesh(core_axis_name="c", subcore_axis_name="s")

    @functools.partial(
        pl.kernel,
        out_type=(
            jax.ShapeDtypeStruct((_NTOK, D), jnp.float32),
            jax.ShapeDtypeStruct((_NUSR_PAD, D), jnp.float32),
        ),
        mesh=mesh,
        scratch_types=[
            pltpu.VMEM((_NW_CHUNKS * _CH,), jnp.int32),
            pltpu.VMEM((_NU_CHUNKS * _CH,), jnp.int32),
            pltpu.VMEM((_NBUF, _CH, D), jnp.float32),
            [pltpu.SemaphoreType.DMA] * _NBUF,
            [pltpu.SemaphoreType.DMA] * _NBUF,
        ],
    )
    def k(wids, uids, wtab, utab, wout, uout, widx_v, uidx_v, rows_v,
          sem_g, sem_w):
        w = lax.axis_index("s") * _NC + lax.axis_index("c")
        wbase = w * _NW_CHUNKS * _CH
        ubase = w * _NU_CHUNKS * _CH
        pltpu.sync_copy(wids.at[pl.ds(wbase, _NW_CHUNKS * _CH)], widx_v)
        pltpu.sync_copy(uids.at[pl.ds(ubase, _NU_CHUNKS * _CH)], uidx_v)

        def group(g, carry):
            handles = []
            for j in range(_NBUF):
                @pl.when(g > 0)
                def _():
                    pltpu.make_async_copy(
                        rows_v.at[j], wout.at[pl.ds(0, _CH)], sem_w[j]).wait()
                idx = widx_v.at[pl.ds((g * _NBUF + j) * _CH, _CH)]
                handles.append(
                    pltpu.async_copy(wtab.at[idx], rows_v.at[j], sem_g[j]))
            for j in range(_NBUF):
                handles[j].wait()
                pltpu.async_copy(
                    rows_v.at[j],
                    wout.at[pl.ds(wbase + (g * _NBUF + j) * _CH, _CH)],
                    sem_w[j])
            return carry

        lax.fori_loop(0, _NGRP, group, 0)
        for j in range(_NBUF):
            pltpu.make_async_copy(
                rows_v.at[j], wout.at[pl.ds(0, _CH)], sem_w[j]).wait()

        uhandles = []
        for j in range(_NU_CHUNKS):
            idx = uidx_v.at[pl.ds(j * _CH, _CH)]
            uhandles.append(
                pltpu.async_copy(utab.at[idx], rows_v.at[j], sem_g[j]))
        for j in range(_NU_CHUNKS):
            uhandles[j].wait()
            pltpu.async_copy(
                rows_v.at[j], uout.at[pl.ds(ubase + j * _CH, _CH)], sem_w[j])
        for j in range(_NU_CHUNKS):
            pltpu.make_async_copy(
                rows_v.at[j], uout.at[pl.ds(0, _CH)], sem_w[j]).wait()

    return k(word_ids, user_ids, word_tab, user_tab)


def _lstm_maxpool_all(x, w_cat, b):
    m = x.shape[0]
    grid = m // _MB

    def body(x_ref, w_ref, b_ref, out_ref):
        wc = w_ref[...]
        bb = b_ref[...]
        h = jnp.zeros((_MB, H), jnp.float32)
        c = jnp.zeros((_MB, H), jnp.float32)
        hm = jnp.full((_MB, H), -jnp.inf, jnp.float32)
        for t in range(LSEQ):
            xt = x_ref[:, t, :]
            xh = jnp.concatenate([xt, h], axis=1)
            g = jnp.dot(xh, wc, preferred_element_type=jnp.float32) + bb
            i = jax.nn.sigmoid(g[:, :H])
            f = jax.nn.sigmoid(g[:, H:2 * H])
            gg = jnp.tanh(g[:, 2 * H:3 * H])
            o = jax.nn.sigmoid(g[:, 3 * H:])
            c = f * c + i * gg
            h = o * jnp.tanh(c)
            hm = jnp.maximum(hm, h)
        out_ref[...] = hm

    return pl.pallas_call(
        body,
        grid=(grid,),
        in_specs=[
            pl.BlockSpec((_MB, LSEQ, D), lambda i: (i, 0, 0)),
            pl.BlockSpec((D + H, 4 * H), lambda i: (0, 0)),
            pl.BlockSpec((1, 4 * H), lambda i: (0, 0)),
        ],
        out_specs=pl.BlockSpec((_MB, H), lambda i: (i, 0)),
        out_shape=jax.ShapeDtypeStruct((m, H), jnp.float32),
    )(x, w_cat, b)


def _second_layer(qsl3, qesl3, usl3, uesl3, qfl, ufl,
                  We, Ws, Wd, be, Wn_q, Wea_q, Wsa_q, b_agg_q,
                  Wn_u, Wea_u, Wsa_u, b_agg_u):
    r = qfl.shape[0]
    scale = 1.0 / float(H) ** 0.5

    def body(qsl_r, qesl_r, usl_r, uesl_r, qfl_r, ufl_r,
             we_r, ws_r, wd_r, be_r, wnq_r, weaq_r, wsaq_r, bq_r,
             wnu_r, weau_r, wsau_r, bu_r,
             qout_r, uout_r, msgq_s, msgu_s):
        dot = lambda a, m: jnp.dot(a, m, preferred_element_type=jnp.float32)
        qfl2 = qfl_r[...]
        ufl2 = ufl_r[...]
        we = we_r[...]
        ws = ws_r[...]
        wd = wd_r[...]
        be2 = be_r[...]
        ufl_wd = dot(ufl2, wd) + be2
        qfl_ws = dot(qfl2, ws) + be2
        wed = we + wd
        sq_list = []
        su_list = []
        for j in range(N2):
            qe = jnp.tanh(dot(qesl_r[:, j, :], we) + dot(qsl_r[:, j, :], ws)
                          + ufl_wd)
            ue = jnp.tanh(dot(uesl_r[:, j, :], wed) + qfl_ws)
            mq = jnp.tanh(dot(usl_r[:, j, :], wnq_r[...]) + dot(qe, weaq_r[...]))
            mu = jnp.tanh(dot(qsl_r[:, j, :], wnu_r[...]) + dot(ue, weau_r[...]))
            msgq_s[:, j, :] = mq
            msgu_s[:, j, :] = mu
            sq_list.append(jnp.sum(qfl2 * mq, axis=1, keepdims=True))
            su_list.append(jnp.sum(ufl2 * mu, axis=1, keepdims=True))

        def softmax(s):
            mx = jnp.max(s, axis=1, keepdims=True)
            e = jnp.exp(s - mx)
            return e / jnp.sum(e, axis=1, keepdims=True)

        aq = softmax(jnp.concatenate(sq_list, axis=1) * scale)
        au = softmax(jnp.concatenate(su_list, axis=1) * scale)
        aggq = jnp.zeros((r, H), jnp.float32)
        aggu = jnp.zeros((r, H), jnp.float32)
        for j in range(N2):
            aggq = aggq + aq[:, j:j + 1] * msgq_s[:, j, :]
            aggu = aggu + au[:, j:j + 1] * msgu_s[:, j, :]
        qout_r[...] = jnp.tanh(dot(qfl2, wsaq_r[...]) + aggq + bq_r[...])
        uout_r[...] = jnp.tanh(dot(ufl2, wsau_r[...]) + aggu + bu_r[...])

    return pl.pallas_call(
        body,
        out_shape=(jax.ShapeDtypeStruct((r, H), jnp.float32),
                   jax.ShapeDtypeStruct((r, H), jnp.float32)),
        scratch_shapes=[pltpu.VMEM((r, N2, H), jnp.float32),
                        pltpu.VMEM((r, N2, H), jnp.float32)],
    )(qsl3, qesl3, usl3, uesl3, qfl, ufl,
      We, Ws, Wd, be, Wn_q, Wea_q, Wsa_q, b_agg_q,
      Wn_u, Wea_u, Wsa_u, b_agg_u)


def _first_layer(qefl3, uefl3, ufl3, qfl3, q1, a1, u1,
                 We, Ws, Wd, be, Wn_q, Wea_q, Wsa_q, b_agg_q,
                 Wn_u, Wea_u, Wsa_u, b_agg_u,
                 wq, wa, wu, bq, ba, bu, wfinal, bfinal):
    scale = 1.0 / float(H) ** 0.5

    def body(qefl_r, uefl_r, ufl_r, qfl_r, q1_r, a1_r, u1_r,
             we_r, ws_r, wd_r, be_r, wnq_r, weaq_r, wsaq_r, bgq_r,
             wnu_r, weau_r, wsau_r, bgu_r,
             wq_r, wa_r, wu_r, bq_r, ba_r, bu_r, wf_r, bf_r,
             res_r, pred_r, msgq_s, msgu_s):
        dot = lambda a, m: jnp.dot(a, m, preferred_element_type=jnp.float32)
        q1v = q1_r[...]
        a1v = a1_r[...]
        u1v = u1_r[...]
        we = we_r[...]
        ws = ws_r[...]
        wd = wd_r[...]
        be2 = be_r[...]
        q1_ws = dot(q1v, ws) + be2
        u1_wd = dot(u1v, wd) + be2
        sq_list = []
        su_list = []
        for j in range(N1):
            uflj = ufl_r[:, j, :]
            qflj = qfl_r[:, j, :]
            qe = jnp.tanh(dot(qefl_r[:, j, :], we) + q1_ws + dot(uflj, wd))
            ue = jnp.tanh(dot(uefl_r[:, j, :], we) + dot(qflj, ws) + u1_wd)
            mq = jnp.tanh(dot(uflj, wnq_r[...]) + dot(qe, weaq_r[...]))
            mu = jnp.tanh(dot(qflj, wnu_r[...]) + dot(ue, weau_r[...]))
            msgq_s[:, j, :] = mq
            msgu_s[:, j, :] = mu
            sq_list.append(jnp.sum(q1v * mq, axis=1, keepdims=True))
            su_list.append(jnp.sum(u1v * mu, axis=1, keepdims=True))

        def softmax(s):
            mx = jnp.max(s, axis=1, keepdims=True)
            e = jnp.exp(s - mx)
            return e / jnp.sum(e, axis=1, keepdims=True)

        aq = softmax(jnp.concatenate(sq_list, axis=1) * scale)
        au = softmax(jnp.concatenate(su_list, axis=1) * scale)
        aggq = jnp.zeros((B, H), jnp.float32)
        aggu = jnp.zeros((B, H), jnp.float32)
        for j in range(N1):
            aggq = aggq + aq[:, j:j + 1] * msgq_s[:, j, :]
            aggu = aggu + au[:, j:j + 1] * msgu_s[:, j, :]
        q1n = jnp.tanh(dot(q1v, wsaq_r[...]) + aggq + bgq_r[...])
        u1n = jnp.tanh(dot(u1v, wsau_r[...]) + aggu + bgu_r[...])
        a1n = jnp.tanh(dot(a1v, we) + dot(q1n, ws) + dot(u1n, wd) + be2)
        t = jnp.tanh(dot(q1n, wq_r[...]) + bq_r[...]
                     + dot(a1n, wa_r[...]) + ba_r[...]
                     + dot(u1n, wu_r[...]) + bu_r[...])
        z = dot(t, wf_r[...]) + bf_r[...]
        mx = jnp.max(z, axis=1, keepdims=True)
        lse = mx + jnp.log(jnp.sum(jnp.exp(z - mx), axis=1, keepdims=True))
        res_r[...] = z - lse
        pred_r[...] = (z[:, 1:2] > z[:, 0:1]).astype(jnp.int32)

    return pl.pallas_call(
        body,
        out_shape=(jax.ShapeDtypeStruct((B, 2), jnp.float32),
                   jax.ShapeDtypeStruct((B, 1), jnp.int32)),
        scratch_shapes=[pltpu.VMEM((B, N1, H), jnp.float32),
                        pltpu.VMEM((B, N1, H), jnp.float32)],
    )(qefl3, uefl3, ufl3, qfl3, q1, a1, u1,
      We, Ws, Wd, be, Wn_q, Wea_q, Wsa_q, b_agg_q,
      Wn_u, Wea_u, Wsa_u, b_agg_u,
      wq, wa, wu, bq, ba, bu, wfinal, bfinal)


def kernel(question1, answer1, user1, score1, question_first_layer,
           question_edge_first_layer, question_edge_score_list_first_layer,
           question_second_layer, question_edge_second_layer,
           question_edge_score_list_second_layer, user_first_layer,
           user_edge_first_layer, user_edge_score_list_first_layer,
           user_second_layer, user_edge_second_layer,
           user_edge_score_list_second_layer, word_embed, user_embed,
           Wih, Whh, b_lstm, We, Ws, Wd, be, Wn_q, Wea_q, Wsa_q, b_agg_q,
           Wn_u, Wea_u, Wsa_u, b_agg_u, wq, wa, wu, bq, ba, bu,
           wfinal, bfinal):
    i32 = jnp.int32
    wids = jnp.concatenate([
        question1.reshape(-1, LSEQ), answer1.reshape(-1, LSEQ),
        user_edge_first_layer.reshape(-1, LSEQ),
        user_edge_second_layer.reshape(-1, LSEQ),
        question_first_layer.reshape(-1, LSEQ),
        question_edge_first_layer.reshape(-1, LSEQ),
        question_second_layer.reshape(-1, LSEQ),
        question_edge_second_layer.reshape(-1, LSEQ),
    ], axis=0).reshape(-1).astype(i32)
    wids = jnp.concatenate(
        [wids, jnp.zeros((_NTOK - _M_TOT * LSEQ,), i32)])
    uids = jnp.concatenate([
        user1.reshape(-1), user_first_layer.reshape(-1),
        user_second_layer.reshape(-1)]).astype(i32)
    uids = jnp.concatenate([uids, jnp.zeros((_NUSR_PAD - _NUSR,), i32)])

    xw, xu = _sc_gather(wids, uids, word_embed, user_embed)

    w_cat = jnp.concatenate([Wih, Whh], axis=0)
    hs = _lstm_maxpool_all(xw.reshape(_M_PAD, LSEQ, D), w_cat,
                           b_lstm.reshape(1, 4 * H))

    o = 0
    q1 = hs[o:o + B]; o += B
    a1 = hs[o:o + B]; o += B
    uefl = hs[o:o + B * N1]; o += B * N1
    uesl = hs[o:o + B * N1 * N2]; o += B * N1 * N2
    qfl = hs[o:o + B * N1]; o += B * N1
    qefl = hs[o:o + B * N1]; o += B * N1
    qsl = hs[o:o + B * N1 * N2]; o += B * N1 * N2
    qesl = hs[o:o + B * N1 * N2]; o += B * N1 * N2

    u1 = xu[:B]
    ufl = xu[B:B + B * N1]
    usl = xu[B + B * N1:_NUSR]

    r3 = lambda a: a.reshape(B * N1, N2, H)
    be2 = be.reshape(1, H)
    qfl_new, ufl_new = _second_layer(
        r3(qsl), r3(qesl), r3(usl), r3(uesl), qfl, ufl,
        We, Ws, Wd, be2, Wn_q, Wea_q, Wsa_q, b_agg_q.reshape(1, H),
        Wn_u, Wea_u, Wsa_u, b_agg_u.reshape(1, H))

    f3 = lambda a: a.reshape(B, N1, H)
    result, predict = _first_layer(
        f3(qefl), f3(uefl), f3(ufl_new), f3(qfl_new), q1, a1, u1,
        We, Ws, Wd, be2, Wn_q, Wea_q, Wsa_q, b_agg_q.reshape(1, H),
        Wn_u, Wea_u, Wsa_u, b_agg_u.reshape(1, H),
        wq, wa, wu, bq.reshape(1, H), ba.reshape(1, H), bu.reshape(1, H),
        wfinal, bfinal.reshape(1, 2))

    return (result, predict.reshape(B))
